# Initial kernel scaffold; baseline (speedup 1.0000x reference)
import jax
import jax.numpy as jnp
from jax.experimental import pallas as pl


def kernel(enc_input, src_emb, pe_table):
    raise NotImplementedError("write your pallas kernel here")


# speedup vs baseline: 1.1668x; 1.1668x over previous
import functools

import jax
import jax.numpy as jnp
from jax import lax
from jax.experimental import pallas as pl
from jax.experimental.pallas import tpu as pltpu
from jax.experimental.pallas import tpu_sc as plsc

PADDING_ID = 0
NUM_CORES = 2
NUM_SUBCORES = 16
LANES = 16
NUM_WORKERS = NUM_CORES * NUM_SUBCORES


@functools.cache
def _build(n_flat: int, seq_len: int, d: int, chunk: int):
    assert n_flat % (NUM_WORKERS * chunk) == 0
    assert chunk % LANES == 0 and d % LANES == 0
    per_worker = n_flat // NUM_WORKERS
    n_chunks = per_worker // chunk
    assert seq_len % per_worker == 0 or per_worker % seq_len == 0

    mesh = plsc.VectorSubcoreMesh(core_axis_name="c", subcore_axis_name="s")

    @functools.partial(
        pl.kernel,
        out_type=jax.ShapeDtypeStruct((n_flat, d), jnp.float32),
        mesh=mesh,
        scratch_types=[
            pltpu.VMEM((chunk,), jnp.int32),
            pltpu.VMEM((chunk,), jnp.int32),
            pltpu.VMEM((chunk, d), jnp.float32),
            pltpu.VMEM((chunk, d), jnp.float32),
            pltpu.SemaphoreType.DMA,
            pltpu.SemaphoreType.DMA,
        ],
    )
    def k(enc_hbm, emb_hbm, pe_hbm, out_hbm, ids_v, pos_v, tok_v, pe_v, s1, s2):
        wid = lax.axis_index("s") * NUM_CORES + lax.axis_index("c")
        base = wid * per_worker

        for ci in range(n_chunks):
            off = base + ci * chunk
            l0 = lax.rem(off, seq_len)
            pltpu.sync_copy(enc_hbm.at[pl.ds(off, chunk)], ids_v)
            for j in range(chunk // LANES):
                sl = pl.ds(j * LANES, LANES)
                ids = ids_v[sl]
                p = lax.broadcasted_iota(jnp.int32, (LANES,), 0) + (l0 + (j * LANES + 1))
                pos_v[sl] = jnp.where(ids == PADDING_ID, 0, p)
            cp_tok = pltpu.async_copy(emb_hbm.at[ids_v], tok_v, s1)
            cp_pe = pltpu.async_copy(pe_hbm.at[pos_v], pe_v, s2)
            cp_tok.wait()
            cp_pe.wait()

            def row_add(i, carry):
                for j in range(d // LANES):
                    sl = pl.ds(j * LANES, LANES)
                    tok_v[i, sl] = tok_v[i, sl] + pe_v[i, sl]
                return carry

            lax.fori_loop(0, chunk, row_add, 0)
            pltpu.sync_copy(tok_v, out_hbm.at[pl.ds(off, chunk)])

    return k


def kernel(enc_input, src_emb, pe_table):
    b, l = enc_input.shape
    d = src_emb.shape[1]
    flat = enc_input.reshape(b * l)
    out = _build(b * l, l, d, 32)(flat, src_emb, pe_table)
    return out.reshape(b, l, d)


# speedup vs baseline: 1.4460x; 1.2393x over previous
import functools

import jax
import jax.numpy as jnp
from jax import lax
from jax.experimental import pallas as pl
from jax.experimental.pallas import tpu as pltpu
from jax.experimental.pallas import tpu_sc as plsc

PADDING_ID = 0
NUM_CORES = 2
NUM_SUBCORES = 16
LANES = 16
NUM_WORKERS = NUM_CORES * NUM_SUBCORES
NBUF = 2


@functools.cache
def _build(n_flat: int, seq_len: int, d: int, chunk: int):
    assert n_flat % (NUM_WORKERS * chunk) == 0
    assert chunk % LANES == 0 and d % LANES == 0
    per_worker = n_flat // NUM_WORKERS
    n_chunks = per_worker // chunk
    assert n_chunks >= 2

    mesh = plsc.VectorSubcoreMesh(core_axis_name="c", subcore_axis_name="s")

    scratch = (
        [pltpu.VMEM((chunk,), jnp.int32) for _ in range(NBUF)]
        + [pltpu.VMEM((chunk,), jnp.int32) for _ in range(NBUF)]
        + [pltpu.VMEM((chunk, d), jnp.float32) for _ in range(NBUF)]
        + [pltpu.VMEM((chunk, d), jnp.float32) for _ in range(NBUF)]
        + [pltpu.SemaphoreType.DMA] * (3 * NBUF)
    )

    @functools.partial(
        pl.kernel,
        out_type=jax.ShapeDtypeStruct((n_flat, d), jnp.float32),
        mesh=mesh,
        scratch_types=scratch,
    )
    def k(enc_hbm, emb_hbm, pe_hbm, out_hbm, *bufs):
        ids_v = bufs[0:NBUF]
        pos_v = bufs[NBUF : 2 * NBUF]
        tok_v = bufs[2 * NBUF : 3 * NBUF]
        pe_v = bufs[3 * NBUF : 4 * NBUF]
        sems = bufs[4 * NBUF :]
        tok_sem = sems[0:NBUF]
        pe_sem = sems[NBUF : 2 * NBUF]
        st_sem = sems[2 * NBUF : 3 * NBUF]

        wid = lax.axis_index("s") * NUM_CORES + lax.axis_index("c")
        base = wid * per_worker

        def issue_gather(ci):
            b = ci % NBUF
            off = base + ci * chunk
            l0 = lax.rem(off, seq_len)
            pltpu.sync_copy(enc_hbm.at[pl.ds(off, chunk)], ids_v[b])
            for j in range(chunk // LANES):
                sl = pl.ds(j * LANES, LANES)
                ids = ids_v[b][sl]
                p = lax.broadcasted_iota(jnp.int32, (LANES,), 0) + (
                    l0 + (j * LANES + 1)
                )
                pos_v[b][sl] = jnp.where(ids == PADDING_ID, 0, p)
            cp_t = pltpu.async_copy(emb_hbm.at[ids_v[b]], tok_v[b], tok_sem[b])
            cp_p = pltpu.async_copy(pe_hbm.at[pos_v[b]], pe_v[b], pe_sem[b])
            return cp_t, cp_p

        gathers = [None] * n_chunks
        stores = [None] * n_chunks
        gathers[0] = issue_gather(0)

        for ci in range(n_chunks):
            b = ci % NBUF
            if ci + 1 < n_chunks:
                if stores[ci + 1 - NBUF] is not None:
                    stores[ci + 1 - NBUF].wait()
                gathers[ci + 1] = issue_gather(ci + 1)
            gathers[ci][0].wait()
            gathers[ci][1].wait()

            def row_add(i, carry, _b=b):
                for j in range(d // LANES):
                    sl = pl.ds(j * LANES, LANES)
                    tok_v[_b][i, sl] = tok_v[_b][i, sl] + pe_v[_b][i, sl]
                return carry

            lax.fori_loop(0, chunk, row_add, 0)
            off = base + ci * chunk
            stores[ci] = pltpu.async_copy(
                tok_v[b], out_hbm.at[pl.ds(off, chunk)], st_sem[b]
            )

        for ci in range(n_chunks - NBUF, n_chunks):
            stores[ci].wait()

    return k


def kernel(enc_input, src_emb, pe_table):
    b, l = enc_input.shape
    d = src_emb.shape[1]
    flat = enc_input.reshape(b * l)
    out = _build(b * l, l, d, 16)(flat, src_emb, pe_table)
    return out.reshape(b, l, d)
